# baseline (speedup 1.0000x reference)
import functools

import jax
import jax.numpy as jnp
from jax import lax
from jax.experimental import pallas as pl
from jax.experimental.pallas import tpu as pltpu
from jax.experimental.pallas import tpu_sc as plsc

N_NODES = 10000
N_EDGES = 320000
D = 128

NC = 2
NS = 16
NW = NC * NS
E_PAD = 327680
EDGES_PER_WORKER = E_PAD // NW
CHUNK = 80
NCHUNKS = EDGES_PER_WORKER // CHUNK
N_ACC = N_NODES + 8

ROW_BLK = 1000
GRID = N_NODES // ROW_BLK

_mesh = plsc.VectorSubcoreMesh(
    core_axis_name="c", subcore_axis_name="s", num_cores=NC, num_subcores=NS
)


NB = 3


def _sc_agg_body(with_deg, *refs):
    if with_deg:
        (y_hbm, pk_hbm, zrow_hbm, zdeg_hbm,
         agg_out, deg_out,
         pk_v, sb0, sb1, sb2, db0, db1, db2, r0, r1, r2, ones_v, degstg_v,
         agg_sh, deg_sh, g0, g1, g2, s0, s1, s2) = refs
    else:
        (y_hbm, pk_hbm, zrow_hbm,
         agg_out,
         pk_v, sb0, sb1, sb2, db0, db1, db2, r0, r1, r2,
         agg_sh, g0, g1, g2, s0, s1, s2) = refs
    srcb = (sb0, sb1, sb2)
    dstb = (db0, db1, db2)
    rows = (r0, r1, r2)
    gsem = (g0, g1, g2)
    ssem = (s0, s1, s2)

    c = lax.axis_index("c")
    s = lax.axis_index("s")
    wid = c * NS + s

    @pl.when(s < 10)
    def _zero():
        pltpu.sync_copy(zrow_hbm, agg_sh.at[pl.ds(s * 1000, 1000)])
        if with_deg:
            pltpu.sync_copy(zdeg_hbm, degstg_v)
            pltpu.sync_copy(degstg_v, deg_sh.at[pl.ds(s * 1000, 1000)])

    if with_deg:
        for i in range(CHUNK // 16):
            ones_v[pl.ds(i * 16, 16)] = jnp.ones((16,), jnp.float32)

    off = pl.multiple_of(wid * EDGES_PER_WORKER, 8)
    pltpu.sync_copy(pk_hbm.at[pl.ds(off, EDGES_PER_WORKER)], pk_v)

    plsc.subcore_barrier()

    def unpack(chunk, b):
        base = chunk * CHUNK
        for k in range(CHUNK // 16):
            p = pk_v[pl.ds(base + 16 * k, 16)]
            srcb[b][pl.ds(16 * k, 16)] = p & 0xFFFF
            dstb[b][pl.ds(16 * k, 16)] = lax.shift_right_logical(p, 16)

    def fire_gather(chunk, b):
        unpack(chunk, b)
        pltpu.async_copy(y_hbm.at[srcb[b]], rows[b], gsem[b])

    def wait_gather(b):
        pltpu.make_async_copy(y_hbm.at[pl.ds(0, CHUNK)], rows[b],
                              gsem[b]).wait()

    def fire_scatter(b):
        pltpu.async_copy(rows[b], agg_sh.at[dstb[b]], ssem[b], add=True)
        if with_deg:
            pltpu.async_copy(ones_v, deg_sh.at[dstb[b]], ssem[b], add=True)

    def wait_scatter(b):
        pltpu.make_async_copy(y_hbm.at[pl.ds(0, CHUNK)], rows[b],
                              ssem[b]).wait()
        if with_deg:
            pltpu.make_async_copy(zrow_hbm.at[0, pl.ds(0, CHUNK)], ones_v,
                                  ssem[b]).wait()

    assert NCHUNKS % NB == 2
    fire_gather(0, 0)

    def body(j, carry):
        for b in range(NB):
            i = NB * j + b
            b1 = (b + 1) % NB

            @pl.when(i >= 2)
            def _w():
                wait_scatter(b1)

            fire_gather(i + 1, b1)
            wait_gather(b)
            fire_scatter(b)
        return carry

    lax.fori_loop(0, NCHUNKS // NB, body, 0)
    wait_scatter(1)
    fire_gather(NCHUNKS - 1, 1)
    wait_gather(0)
    fire_scatter(0)
    wait_gather(1)
    fire_scatter(1)
    wait_scatter(2)
    wait_scatter(0)
    wait_scatter(1)

    plsc.subcore_barrier()

    @pl.when(s < 10)
    def _writeback():
        pltpu.sync_copy(agg_sh.at[pl.ds(s * 1000, 1000)],
                        agg_out.at[c, pl.ds(s * 1000, 1000)])
        if with_deg:
            off = pl.multiple_of(c * N_NODES + s * 1000, 8)
            pltpu.sync_copy(deg_sh.at[pl.ds(s * 1000, 1000)], degstg_v)
            pltpu.sync_copy(degstg_v, deg_out.at[pl.ds(off, 1000)])


_sc_agg_deg = pl.kernel(
    functools.partial(_sc_agg_body, True),
    out_type=(
        jax.ShapeDtypeStruct((NC, N_NODES, D), jnp.float32),
        jax.ShapeDtypeStruct((NC * N_NODES,), jnp.float32),
    ),
    mesh=_mesh,
    scratch_types=(
        pltpu.VMEM((EDGES_PER_WORKER,), jnp.int32),
        pltpu.VMEM((CHUNK,), jnp.int32),
        pltpu.VMEM((CHUNK,), jnp.int32),
        pltpu.VMEM((CHUNK,), jnp.int32),
        pltpu.VMEM((CHUNK,), jnp.int32),
        pltpu.VMEM((CHUNK,), jnp.int32),
        pltpu.VMEM((CHUNK,), jnp.int32),
        pltpu.VMEM((CHUNK, D), jnp.float32),
        pltpu.VMEM((CHUNK, D), jnp.float32),
        pltpu.VMEM((CHUNK, D), jnp.float32),
        pltpu.VMEM((CHUNK,), jnp.float32),
        pltpu.VMEM((1000,), jnp.float32),
        pltpu.VMEM_SHARED((N_ACC, D), jnp.float32),
        pltpu.VMEM_SHARED((N_ACC,), jnp.float32),
        pltpu.SemaphoreType.DMA,
        pltpu.SemaphoreType.DMA,
        pltpu.SemaphoreType.DMA,
        pltpu.SemaphoreType.DMA,
        pltpu.SemaphoreType.DMA,
        pltpu.SemaphoreType.DMA,
    ),
)

_sc_agg = pl.kernel(
    functools.partial(_sc_agg_body, False),
    out_type=jax.ShapeDtypeStruct((NC, N_NODES, D), jnp.float32),
    mesh=_mesh,
    scratch_types=(
        pltpu.VMEM((EDGES_PER_WORKER,), jnp.int32),
        pltpu.VMEM((CHUNK,), jnp.int32),
        pltpu.VMEM((CHUNK,), jnp.int32),
        pltpu.VMEM((CHUNK,), jnp.int32),
        pltpu.VMEM((CHUNK,), jnp.int32),
        pltpu.VMEM((CHUNK,), jnp.int32),
        pltpu.VMEM((CHUNK,), jnp.int32),
        pltpu.VMEM((CHUNK, D), jnp.float32),
        pltpu.VMEM((CHUNK, D), jnp.float32),
        pltpu.VMEM((CHUNK, D), jnp.float32),
        pltpu.VMEM_SHARED((N_ACC, D), jnp.float32),
        pltpu.SemaphoreType.DMA,
        pltpu.SemaphoreType.DMA,
        pltpu.SemaphoreType.DMA,
        pltpu.SemaphoreType.DMA,
        pltpu.SemaphoreType.DMA,
        pltpu.SemaphoreType.DMA,
    ),
)


def _tc1_body(x_ref, wl_ref, wr_ref, b_ref, ei_ref, y_ref, z_ref, pk_ref):
    xb = x_ref[...]
    y_ref[...] = jnp.dot(xb, wl_ref[...], preferred_element_type=jnp.float32)
    z_ref[...] = (
        jnp.dot(xb, wr_ref[...], preferred_element_type=jnp.float32)
        + b_ref[...]
    )
    pk_ref[...] = ei_ref[0] | (ei_ref[1] << 16)


def _tc2_body(agg_ref, deg_ref, z_ref, wl_ref, wr_ref, b_ref, y2_ref, z2_ref):
    agg = agg_ref[0] + agg_ref[1]
    deg = jnp.maximum(deg_ref[0] + deg_ref[1], 1.0)
    h = jnp.maximum(agg / deg + z_ref[...], 0.0)
    y2_ref[...] = jnp.dot(h, wl_ref[...], preferred_element_type=jnp.float32)
    z2_ref[...] = (
        jnp.dot(h, wr_ref[...], preferred_element_type=jnp.float32)
        + b_ref[...]
    )


def _tc3_body(agg_ref, deg_ref, z_ref, o_ref):
    agg = agg_ref[0] + agg_ref[1]
    deg = jnp.maximum(deg_ref[0] + deg_ref[1], 1.0)
    h = agg / deg + z_ref[...]
    m = jnp.max(h, axis=-1, keepdims=True)
    e = jnp.exp(h - m)
    lse = jnp.log(jnp.sum(e, axis=-1, keepdims=True)) + m
    o_ref[...] = h - lse


_row_spec = pl.BlockSpec((ROW_BLK, D), lambda i: (i, 0))
_w_spec = pl.BlockSpec((D, D), lambda i: (0, 0))
_b_spec = pl.BlockSpec((1, D), lambda i: (0, 0))
_agg_spec = pl.BlockSpec((NC, ROW_BLK, D), lambda i: (0, i, 0))
_deg_spec = pl.BlockSpec((NC, ROW_BLK, 1), lambda i: (0, i, 0))

_E_BLK = E_PAD // GRID
_tc1 = pl.pallas_call(
    _tc1_body,
    grid=(GRID,),
    in_specs=[_row_spec, _w_spec, _w_spec, _b_spec,
              pl.BlockSpec((2, _E_BLK), lambda i: (0, i))],
    out_specs=[_row_spec, _row_spec,
               pl.BlockSpec((_E_BLK,), lambda i: (i,))],
    out_shape=[
        jax.ShapeDtypeStruct((N_NODES, D), jnp.float32),
        jax.ShapeDtypeStruct((N_NODES, D), jnp.float32),
        jax.ShapeDtypeStruct((E_PAD,), jnp.int32),
    ],
)

_tc2 = pl.pallas_call(
    _tc2_body,
    grid=(GRID,),
    in_specs=[_agg_spec, _deg_spec, _row_spec, _w_spec, _w_spec, _b_spec],
    out_specs=[_row_spec, _row_spec],
    out_shape=[
        jax.ShapeDtypeStruct((N_NODES, D), jnp.float32),
        jax.ShapeDtypeStruct((N_NODES, D), jnp.float32),
    ],
)

_tc3 = pl.pallas_call(
    _tc3_body,
    grid=(GRID,),
    in_specs=[_agg_spec, _deg_spec, _row_spec],
    out_specs=_row_spec,
    out_shape=jax.ShapeDtypeStruct((N_NODES, D), jnp.float32),
)


def kernel(x, edge_index, W1_l, W1_r, b1, W2_l, W2_r, b2):
    ei = edge_index.astype(jnp.int32)
    pad = jnp.concatenate(
        [jnp.zeros((1, E_PAD - N_EDGES), jnp.int32),
         jnp.full((1, E_PAD - N_EDGES), N_NODES, jnp.int32)], axis=0)
    ei = jnp.concatenate([ei, pad], axis=1)
    zrow = jnp.zeros((1000, D), jnp.float32)
    zdeg = jnp.zeros((1000,), jnp.float32)
    b1r = b1.reshape(1, D)
    b2r = b2.reshape(1, D)

    y1, z1, pk = _tc1(x, W1_l, W1_r, b1r, ei)
    agg1, deg = _sc_agg_deg(y1, pk, zrow, zdeg)
    deg3 = deg.reshape(NC, N_NODES, 1)
    y2, z2 = _tc2(agg1, deg3, z1, W2_l, W2_r, b2r)
    agg2 = _sc_agg(y2, pk, zrow)
    out = _tc3(agg2, deg3, z2)
    return out


# speedup vs baseline: 1.0414x; 1.0414x over previous
import functools

import jax
import jax.numpy as jnp
from jax import lax
from jax.experimental import pallas as pl
from jax.experimental.pallas import tpu as pltpu
from jax.experimental.pallas import tpu_sc as plsc

N_NODES = 10000
N_EDGES = 320000
D = 128

NC = 2
NS = 16
NW = NC * NS
E_PAD = 327680
EDGES_PER_WORKER = E_PAD // NW
CHUNK = 80
NCHUNKS = EDGES_PER_WORKER // CHUNK
PAD_CHUNK0 = (N_EDGES - (NW - 1) * EDGES_PER_WORKER) // CHUNK
N_Y = 11000

ROW_BLK = 1000
GRID = N_NODES // ROW_BLK
GRID_Y = GRID + 1

_mesh = plsc.VectorSubcoreMesh(
    core_axis_name="c", subcore_axis_name="s", num_cores=NC, num_subcores=NS
)


NB = 3


def _sc_agg_body(with_deg, *refs):
    if with_deg:
        (y_hbm, pk_hbm, zrow_hbm, zdeg_hbm,
         agg_out, deg_out,
         pk_v, sb0, sb1, sb2, db0, db1, db2, r0, r1, r2, ones_v, degstg_v,
         agg_sh, deg_sh, g0, g1, g2, s0, s1, s2) = refs
    else:
        (y_hbm, pk_hbm, zrow_hbm,
         agg_out,
         pk_v, sb0, sb1, sb2, db0, db1, db2, r0, r1, r2,
         agg_sh, g0, g1, g2, s0, s1, s2) = refs
    srcb = (sb0, sb1, sb2)
    dstb = (db0, db1, db2)
    rows = (r0, r1, r2)
    gsem = (g0, g1, g2)
    ssem = (s0, s1, s2)

    c = lax.axis_index("c")
    s = lax.axis_index("s")
    wid = c * NS + s

    @pl.when(s < 10)
    def _zero():
        pltpu.sync_copy(zrow_hbm, agg_sh.at[pl.ds(s * 1000, 1000)])
        if with_deg:
            pltpu.sync_copy(zdeg_hbm, degstg_v)
            pltpu.sync_copy(degstg_v, deg_sh.at[pl.ds(s * 1000, 1000)])

    if with_deg:
        for i in range(CHUNK // 16):
            ones_v[pl.ds(i * 16, 16)] = jnp.ones((16,), jnp.float32)

    off = pl.multiple_of(wid * EDGES_PER_WORKER, 8)
    pltpu.sync_copy(pk_hbm.at[pl.ds(off, EDGES_PER_WORKER)], pk_v)

    plsc.subcore_barrier()

    def unpack(chunk, b):
        base = chunk * CHUNK
        for k in range(CHUNK // 16):
            p = pk_v[pl.ds(base + 16 * k, 16)]
            srcb[b][pl.ds(16 * k, 16)] = p & 0xFFFF
            dstb[b][pl.ds(16 * k, 16)] = lax.shift_right_logical(p, 16)

    def fire_gather(chunk, b):
        unpack(chunk, b)
        pltpu.async_copy(y_hbm.at[srcb[b]], rows[b], gsem[b])

    def wait_gather(b):
        pltpu.make_async_copy(y_hbm.at[pl.ds(0, CHUNK)], rows[b],
                              gsem[b]).wait()

    is_last = wid == (NW - 1)

    def fire_scatter(chunk, b):
        pltpu.async_copy(rows[b], agg_sh.at[dstb[b]], ssem[b], add=True)
        if with_deg:
            @pl.when(jnp.logical_not(is_last & (chunk >= PAD_CHUNK0)))
            def _():
                pltpu.async_copy(ones_v, deg_sh.at[dstb[b]], ssem[b],
                                 add=True)

    def wait_scatter(chunk, b):
        pltpu.make_async_copy(y_hbm.at[pl.ds(0, CHUNK)], rows[b],
                              ssem[b]).wait()
        if with_deg:
            @pl.when(jnp.logical_not(is_last & (chunk >= PAD_CHUNK0)))
            def _():
                pltpu.make_async_copy(zrow_hbm.at[0, pl.ds(0, CHUNK)],
                                      ones_v, ssem[b]).wait()

    assert NCHUNKS % NB == 2
    fire_gather(0, 0)

    def body(j, carry):
        for b in range(NB):
            i = NB * j + b
            b1 = (b + 1) % NB

            @pl.when(i >= 2)
            def _w():
                wait_scatter(i - 2, b1)

            fire_gather(i + 1, b1)
            wait_gather(b)
            fire_scatter(i, b)
        return carry

    lax.fori_loop(0, NCHUNKS // NB, body, 0)
    wait_scatter(NCHUNKS - 4, 1)
    fire_gather(NCHUNKS - 1, 1)
    wait_gather(0)
    fire_scatter(NCHUNKS - 2, 0)
    wait_gather(1)
    fire_scatter(NCHUNKS - 1, 1)
    wait_scatter(NCHUNKS - 3, 2)
    wait_scatter(NCHUNKS - 2, 0)
    wait_scatter(NCHUNKS - 1, 1)

    plsc.subcore_barrier()

    @pl.when(s < 10)
    def _writeback():
        pltpu.sync_copy(agg_sh.at[pl.ds(s * 1000, 1000)],
                        agg_out.at[c, pl.ds(s * 1000, 1000)])
        if with_deg:
            off = pl.multiple_of(c * N_NODES + s * 1000, 8)
            pltpu.sync_copy(deg_sh.at[pl.ds(s * 1000, 1000)], degstg_v)
            pltpu.sync_copy(degstg_v, deg_out.at[pl.ds(off, 1000)])


_sc_agg_deg = pl.kernel(
    functools.partial(_sc_agg_body, True),
    out_type=(
        jax.ShapeDtypeStruct((NC, N_NODES, D), jnp.float32),
        jax.ShapeDtypeStruct((NC * N_NODES,), jnp.float32),
    ),
    mesh=_mesh,
    scratch_types=(
        pltpu.VMEM((EDGES_PER_WORKER,), jnp.int32),
        pltpu.VMEM((CHUNK,), jnp.int32),
        pltpu.VMEM((CHUNK,), jnp.int32),
        pltpu.VMEM((CHUNK,), jnp.int32),
        pltpu.VMEM((CHUNK,), jnp.int32),
        pltpu.VMEM((CHUNK,), jnp.int32),
        pltpu.VMEM((CHUNK,), jnp.int32),
        pltpu.VMEM((CHUNK, D), jnp.float32),
        pltpu.VMEM((CHUNK, D), jnp.float32),
        pltpu.VMEM((CHUNK, D), jnp.float32),
        pltpu.VMEM((CHUNK,), jnp.float32),
        pltpu.VMEM((1000,), jnp.float32),
        pltpu.VMEM_SHARED((N_NODES, D), jnp.float32),
        pltpu.VMEM_SHARED((N_NODES,), jnp.float32),
        pltpu.SemaphoreType.DMA,
        pltpu.SemaphoreType.DMA,
        pltpu.SemaphoreType.DMA,
        pltpu.SemaphoreType.DMA,
        pltpu.SemaphoreType.DMA,
        pltpu.SemaphoreType.DMA,
    ),
)

_sc_agg = pl.kernel(
    functools.partial(_sc_agg_body, False),
    out_type=jax.ShapeDtypeStruct((NC, N_NODES, D), jnp.float32),
    mesh=_mesh,
    scratch_types=(
        pltpu.VMEM((EDGES_PER_WORKER,), jnp.int32),
        pltpu.VMEM((CHUNK,), jnp.int32),
        pltpu.VMEM((CHUNK,), jnp.int32),
        pltpu.VMEM((CHUNK,), jnp.int32),
        pltpu.VMEM((CHUNK,), jnp.int32),
        pltpu.VMEM((CHUNK,), jnp.int32),
        pltpu.VMEM((CHUNK,), jnp.int32),
        pltpu.VMEM((CHUNK, D), jnp.float32),
        pltpu.VMEM((CHUNK, D), jnp.float32),
        pltpu.VMEM((CHUNK, D), jnp.float32),
        pltpu.VMEM_SHARED((N_NODES, D), jnp.float32),
        pltpu.SemaphoreType.DMA,
        pltpu.SemaphoreType.DMA,
        pltpu.SemaphoreType.DMA,
        pltpu.SemaphoreType.DMA,
        pltpu.SemaphoreType.DMA,
        pltpu.SemaphoreType.DMA,
    ),
)


def _tc1_body(x_ref, wl_ref, wr_ref, b_ref, ei_ref, y_ref, z_ref, pk_ref):
    pid = pl.program_id(0)

    @pl.when(pid < GRID)
    def _compute():
        xb = x_ref[...]
        y_ref[...] = jnp.dot(xb, wl_ref[...],
                             preferred_element_type=jnp.float32)
        z_ref[...] = (
            jnp.dot(xb, wr_ref[...], preferred_element_type=jnp.float32)
            + b_ref[...]
        )

    @pl.when(pid == GRID)
    def _zero_row_block():
        y_ref[...] = jnp.zeros((ROW_BLK, D), jnp.float32)
        z_ref[...] = jnp.zeros((ROW_BLK, D), jnp.float32)

    pk_ref[...] = ei_ref[0] | (ei_ref[1] << 16)


def _tc2_body(agg_ref, deg_ref, z_ref, wl_ref, wr_ref, b_ref, y2_ref, z2_ref):
    pid = pl.program_id(0)

    @pl.when(pid < GRID)
    def _compute():
        agg = agg_ref[0] + agg_ref[1]
        deg = jnp.maximum(deg_ref[0] + deg_ref[1], 1.0)
        h = jnp.maximum(agg / deg + z_ref[...], 0.0)
        y2_ref[...] = jnp.dot(h, wl_ref[...],
                              preferred_element_type=jnp.float32)
        z2_ref[...] = (
            jnp.dot(h, wr_ref[...], preferred_element_type=jnp.float32)
            + b_ref[...]
        )

    @pl.when(pid == GRID)
    def _zero_row_block():
        y2_ref[...] = jnp.zeros((ROW_BLK, D), jnp.float32)
        z2_ref[...] = jnp.zeros((ROW_BLK, D), jnp.float32)


def _tc3_body(agg_ref, deg_ref, z_ref, o_ref):
    agg = agg_ref[0] + agg_ref[1]
    deg = jnp.maximum(deg_ref[0] + deg_ref[1], 1.0)
    h = agg / deg + z_ref[...]
    m = jnp.max(h, axis=-1, keepdims=True)
    e = jnp.exp(h - m)
    lse = jnp.log(jnp.sum(e, axis=-1, keepdims=True)) + m
    o_ref[...] = h - lse


_row_spec = pl.BlockSpec((ROW_BLK, D), lambda i: (i, 0))
_row_clamp = pl.BlockSpec((ROW_BLK, D), lambda i: (jnp.minimum(i, GRID - 1), 0))
_w_spec = pl.BlockSpec((D, D), lambda i: (0, 0))
_b_spec = pl.BlockSpec((1, D), lambda i: (0, 0))
_agg_spec = pl.BlockSpec((NC, ROW_BLK, D), lambda i: (0, i, 0))
_agg_clamp = pl.BlockSpec((NC, ROW_BLK, D),
                          lambda i: (0, jnp.minimum(i, GRID - 1), 0))
_deg_spec = pl.BlockSpec((NC, ROW_BLK, 1), lambda i: (0, i, 0))
_deg_clamp = pl.BlockSpec((NC, ROW_BLK, 1),
                          lambda i: (0, jnp.minimum(i, GRID - 1), 0))

_E_BLK = E_PAD // GRID
_tc1 = pl.pallas_call(
    _tc1_body,
    grid=(GRID_Y,),
    in_specs=[_row_clamp, _w_spec, _w_spec, _b_spec,
              pl.BlockSpec((2, _E_BLK),
                           lambda i: (0, jnp.minimum(i, GRID - 1)))],
    out_specs=[_row_spec, _row_spec,
               pl.BlockSpec((_E_BLK,),
                            lambda i: (jnp.minimum(i, GRID - 1),))],
    out_shape=[
        jax.ShapeDtypeStruct((N_Y, D), jnp.float32),
        jax.ShapeDtypeStruct((N_Y, D), jnp.float32),
        jax.ShapeDtypeStruct((E_PAD,), jnp.int32),
    ],
)

_tc2 = pl.pallas_call(
    _tc2_body,
    grid=(GRID_Y,),
    in_specs=[_agg_clamp, _deg_clamp, _row_clamp, _w_spec, _w_spec, _b_spec],
    out_specs=[_row_spec, _row_spec],
    out_shape=[
        jax.ShapeDtypeStruct((N_Y, D), jnp.float32),
        jax.ShapeDtypeStruct((N_Y, D), jnp.float32),
    ],
)

_tc3 = pl.pallas_call(
    _tc3_body,
    grid=(GRID,),
    in_specs=[_agg_spec, _deg_spec, _row_spec],
    out_specs=_row_spec,
    out_shape=jax.ShapeDtypeStruct((N_NODES, D), jnp.float32),
)


def kernel(x, edge_index, W1_l, W1_r, b1, W2_l, W2_r, b2):
    ei = edge_index.astype(jnp.int32)
    pad = jnp.concatenate(
        [jnp.full((1, E_PAD - N_EDGES), N_NODES, jnp.int32),
         jnp.arange(E_PAD - N_EDGES, dtype=jnp.int32)[None, :]], axis=0)
    ei = jnp.concatenate([ei, pad], axis=1)
    zrow = jnp.zeros((1000, D), jnp.float32)
    zdeg = jnp.zeros((1000,), jnp.float32)
    b1r = b1.reshape(1, D)
    b2r = b2.reshape(1, D)

    y1, z1, pk = _tc1(x, W1_l, W1_r, b1r, ei)
    agg1, deg = _sc_agg_deg(y1, pk, zrow, zdeg)
    deg3 = deg.reshape(NC, N_NODES, 1)
    y2, z2 = _tc2(agg1, deg3, z1, W2_l, W2_r, b2r)
    agg2 = _sc_agg(y2, pk, zrow)
    out = _tc3(agg2, deg3, z2)
    return out


# speedup vs baseline: 3.6624x; 3.5167x over previous
import functools

import jax
import jax.numpy as jnp
from jax import lax
from jax.experimental import pallas as pl
from jax.experimental.pallas import tpu as pltpu
from jax.experimental.pallas import tpu_sc as plsc

N_NODES = 10000
N_EDGES = 320000
D = 128

NC = 2
NS = 16
NW = NC * NS
E_PAD = 327680
EDGES_PER_WORKER = E_PAD // NW
CHUNK = 80
NCHUNKS = EDGES_PER_WORKER // CHUNK
PAD_CHUNK0 = (N_EDGES - (NW - 1) * EDGES_PER_WORKER) // CHUNK
N_Y = 11000

ROW_BLK = 1000
GRID = N_NODES // ROW_BLK
GRID_Y = GRID + 1

_mesh = plsc.VectorSubcoreMesh(
    core_axis_name="c", subcore_axis_name="s", num_cores=NC, num_subcores=NS
)


NB = 3


def _sc_agg_body(with_deg, *refs):
    if with_deg:
        (y_hbm, pk_hbm, zrow_hbm, zdeg_hbm,
         agg_out, deg_out,
         pk_v, sb0, sb1, sb2, db0, db1, db2, r0, r1, r2, ones_v, degstg_v,
         agg_sh, deg_sh, g0, g1, g2, s0, s1, s2) = refs
    else:
        (y_hbm, pk_hbm, zrow_hbm,
         agg_out,
         pk_v, sb0, sb1, sb2, db0, db1, db2, r0, r1, r2,
         agg_sh, g0, g1, g2, s0, s1, s2) = refs
    srcb = (sb0, sb1, sb2)
    dstb = (db0, db1, db2)
    rows = (r0, r1, r2)
    gsem = (g0, g1, g2)
    ssem = (s0, s1, s2)

    c = lax.axis_index("c")
    s = lax.axis_index("s")
    wid = c * NS + s

    @pl.when(s < 10)
    def _zero():
        pltpu.sync_copy(zrow_hbm, agg_sh.at[pl.ds(s * 1000, 1000)])
        if with_deg:
            pltpu.sync_copy(zdeg_hbm, degstg_v)
            pltpu.sync_copy(degstg_v, deg_sh.at[pl.ds(s * 1000, 1000)])

    if with_deg:
        for i in range(CHUNK // 16):
            ones_v[pl.ds(i * 16, 16)] = jnp.ones((16,), jnp.float32)

    off = pl.multiple_of(wid * EDGES_PER_WORKER, 8)
    pltpu.sync_copy(pk_hbm.at[pl.ds(off, EDGES_PER_WORKER)], pk_v)

    plsc.subcore_barrier()

    def unpack(chunk, b):
        base = chunk * CHUNK
        for k in range(CHUNK // 16):
            p = pk_v[pl.ds(base + 16 * k, 16)]
            srcb[b][pl.ds(16 * k, 16)] = p & 0xFFFF
            dstb[b][pl.ds(16 * k, 16)] = lax.shift_right_logical(p, 16)

    def fire_gather(chunk, b):
        unpack(chunk, b)
        pltpu.async_copy(y_hbm.at[srcb[b]], rows[b], gsem[b])

    def wait_gather(b):
        pltpu.make_async_copy(y_hbm.at[pl.ds(0, CHUNK)], rows[b],
                              gsem[b]).wait()

    is_last = wid == (NW - 1)

    def fire_scatter(chunk, b):
        pltpu.async_copy(rows[b], agg_sh.at[dstb[b]], ssem[b], add=True)
        if with_deg:
            @pl.when(jnp.logical_not(is_last & (chunk >= PAD_CHUNK0)))
            def _():
                pltpu.async_copy(ones_v, deg_sh.at[dstb[b]], ssem[b],
                                 add=True)

    def wait_scatter(chunk, b):
        pltpu.make_async_copy(y_hbm.at[pl.ds(0, CHUNK)], rows[b],
                              ssem[b]).wait()
        if with_deg:
            @pl.when(jnp.logical_not(is_last & (chunk >= PAD_CHUNK0)))
            def _():
                pltpu.make_async_copy(zrow_hbm.at[0, pl.ds(0, CHUNK)],
                                      ones_v, ssem[b]).wait()

    assert NCHUNKS % NB == 2
    fire_gather(0, 0)

    def body(j, carry):
        for b in range(NB):
            i = NB * j + b
            b1 = (b + 1) % NB

            @pl.when(i >= 2)
            def _w():
                wait_scatter(i - 2, b1)

            fire_gather(i + 1, b1)
            wait_gather(b)
            fire_scatter(i, b)
        return carry

    lax.fori_loop(0, NCHUNKS // NB, body, 0)
    wait_scatter(NCHUNKS - 4, 1)
    fire_gather(NCHUNKS - 1, 1)
    wait_gather(0)
    fire_scatter(NCHUNKS - 2, 0)
    wait_gather(1)
    fire_scatter(NCHUNKS - 1, 1)
    wait_scatter(NCHUNKS - 3, 2)
    wait_scatter(NCHUNKS - 2, 0)
    wait_scatter(NCHUNKS - 1, 1)

    plsc.subcore_barrier()

    @pl.when(s < 10)
    def _writeback():
        pltpu.sync_copy(agg_sh.at[pl.ds(s * 1000, 1000)],
                        agg_out.at[c, pl.ds(s * 1000, 1000)])
        if with_deg:
            off = pl.multiple_of(c * N_NODES + s * 1000, 8)
            pltpu.sync_copy(deg_sh.at[pl.ds(s * 1000, 1000)], degstg_v)
            pltpu.sync_copy(degstg_v, deg_out.at[pl.ds(off, 1000)])


_sc_agg_deg = pl.kernel(
    functools.partial(_sc_agg_body, True),
    out_type=(
        jax.ShapeDtypeStruct((NC, N_NODES, D), jnp.float32),
        jax.ShapeDtypeStruct((NC * N_NODES,), jnp.float32),
    ),
    mesh=_mesh,
    scratch_types=(
        pltpu.VMEM((EDGES_PER_WORKER,), jnp.int32),
        pltpu.VMEM((CHUNK,), jnp.int32),
        pltpu.VMEM((CHUNK,), jnp.int32),
        pltpu.VMEM((CHUNK,), jnp.int32),
        pltpu.VMEM((CHUNK,), jnp.int32),
        pltpu.VMEM((CHUNK,), jnp.int32),
        pltpu.VMEM((CHUNK,), jnp.int32),
        pltpu.VMEM((CHUNK, D), jnp.float32),
        pltpu.VMEM((CHUNK, D), jnp.float32),
        pltpu.VMEM((CHUNK, D), jnp.float32),
        pltpu.VMEM((CHUNK,), jnp.float32),
        pltpu.VMEM((1000,), jnp.float32),
        pltpu.VMEM_SHARED((N_NODES, D), jnp.float32),
        pltpu.VMEM_SHARED((N_NODES,), jnp.float32),
        pltpu.SemaphoreType.DMA,
        pltpu.SemaphoreType.DMA,
        pltpu.SemaphoreType.DMA,
        pltpu.SemaphoreType.DMA,
        pltpu.SemaphoreType.DMA,
        pltpu.SemaphoreType.DMA,
    ),
)

_sc_agg = pl.kernel(
    functools.partial(_sc_agg_body, False),
    out_type=jax.ShapeDtypeStruct((NC, N_NODES, D), jnp.float32),
    mesh=_mesh,
    scratch_types=(
        pltpu.VMEM((EDGES_PER_WORKER,), jnp.int32),
        pltpu.VMEM((CHUNK,), jnp.int32),
        pltpu.VMEM((CHUNK,), jnp.int32),
        pltpu.VMEM((CHUNK,), jnp.int32),
        pltpu.VMEM((CHUNK,), jnp.int32),
        pltpu.VMEM((CHUNK,), jnp.int32),
        pltpu.VMEM((CHUNK,), jnp.int32),
        pltpu.VMEM((CHUNK, D), jnp.float32),
        pltpu.VMEM((CHUNK, D), jnp.float32),
        pltpu.VMEM((CHUNK, D), jnp.float32),
        pltpu.VMEM_SHARED((N_NODES, D), jnp.float32),
        pltpu.SemaphoreType.DMA,
        pltpu.SemaphoreType.DMA,
        pltpu.SemaphoreType.DMA,
        pltpu.SemaphoreType.DMA,
        pltpu.SemaphoreType.DMA,
        pltpu.SemaphoreType.DMA,
    ),
)


def _tc1_body(x_ref, wl_ref, wr_ref, b_ref, ei_ref, y_ref, z_ref, pk_ref):
    pid = pl.program_id(0)

    @pl.when(pid < GRID)
    def _compute():
        xb = x_ref[...]
        y_ref[...] = jnp.dot(xb, wl_ref[...],
                             preferred_element_type=jnp.float32)
        z_ref[...] = (
            jnp.dot(xb, wr_ref[...], preferred_element_type=jnp.float32)
            + b_ref[...]
        )

    @pl.when(pid == GRID)
    def _zero_row_block():
        y_ref[...] = jnp.zeros((ROW_BLK, D), jnp.float32)
        z_ref[...] = jnp.zeros((ROW_BLK, D), jnp.float32)

    pk_ref[...] = ei_ref[0] | (ei_ref[1] << 16)


def _tc2_body(agg_ref, deg_ref, z_ref, wl_ref, wr_ref, b_ref, y2_ref, z2_ref):
    pid = pl.program_id(0)

    @pl.when(pid < GRID)
    def _compute():
        agg = agg_ref[0] + agg_ref[1]
        deg = jnp.maximum(deg_ref[0] + deg_ref[1], 1.0)
        h = jnp.maximum(agg / deg + z_ref[...], 0.0)
        y2_ref[...] = jnp.dot(h, wl_ref[...],
                              preferred_element_type=jnp.float32)
        z2_ref[...] = (
            jnp.dot(h, wr_ref[...], preferred_element_type=jnp.float32)
            + b_ref[...]
        )

    @pl.when(pid == GRID)
    def _zero_row_block():
        y2_ref[...] = jnp.zeros((ROW_BLK, D), jnp.float32)
        z2_ref[...] = jnp.zeros((ROW_BLK, D), jnp.float32)


def _tc3_body(agg_ref, deg_ref, z_ref, o_ref):
    agg = agg_ref[0] + agg_ref[1]
    deg = jnp.maximum(deg_ref[0] + deg_ref[1], 1.0)
    h = agg / deg + z_ref[...]
    m = jnp.max(h, axis=-1, keepdims=True)
    e = jnp.exp(h - m)
    lse = jnp.log(jnp.sum(e, axis=-1, keepdims=True)) + m
    o_ref[...] = h - lse


_row_spec = pl.BlockSpec((ROW_BLK, D), lambda i: (i, 0))
_row_clamp = pl.BlockSpec((ROW_BLK, D), lambda i: (jnp.minimum(i, GRID - 1), 0))
_w_spec = pl.BlockSpec((D, D), lambda i: (0, 0))
_b_spec = pl.BlockSpec((1, D), lambda i: (0, 0))
_agg_spec = pl.BlockSpec((NC, ROW_BLK, D), lambda i: (0, i, 0))
_agg_clamp = pl.BlockSpec((NC, ROW_BLK, D),
                          lambda i: (0, jnp.minimum(i, GRID - 1), 0))
_deg_spec = pl.BlockSpec((NC, ROW_BLK, 1), lambda i: (0, i, 0))
_deg_clamp = pl.BlockSpec((NC, ROW_BLK, 1),
                          lambda i: (0, jnp.minimum(i, GRID - 1), 0))

_E_BLK = E_PAD // GRID
_tc1 = pl.pallas_call(
    _tc1_body,
    grid=(GRID_Y,),
    in_specs=[_row_clamp, _w_spec, _w_spec, _b_spec,
              pl.BlockSpec((2, _E_BLK),
                           lambda i: (0, jnp.minimum(i, GRID - 1)))],
    out_specs=[_row_spec, _row_spec,
               pl.BlockSpec((_E_BLK,),
                            lambda i: (jnp.minimum(i, GRID - 1),))],
    out_shape=[
        jax.ShapeDtypeStruct((N_Y, D), jnp.float32),
        jax.ShapeDtypeStruct((N_Y, D), jnp.float32),
        jax.ShapeDtypeStruct((E_PAD,), jnp.int32),
    ],
)

_tc2 = pl.pallas_call(
    _tc2_body,
    grid=(GRID_Y,),
    in_specs=[_agg_clamp, _deg_clamp, _row_clamp, _w_spec, _w_spec, _b_spec],
    out_specs=[_row_spec, _row_spec],
    out_shape=[
        jax.ShapeDtypeStruct((N_Y, D), jnp.float32),
        jax.ShapeDtypeStruct((N_Y, D), jnp.float32),
    ],
)

_tc3 = pl.pallas_call(
    _tc3_body,
    grid=(GRID,),
    in_specs=[_agg_spec, _deg_spec, _row_spec],
    out_specs=_row_spec,
    out_shape=jax.ShapeDtypeStruct((N_NODES, D), jnp.float32),
)


def kernel(x, edge_index, W1_l, W1_r, b1, W2_l, W2_r, b2):
    ei = edge_index.astype(jnp.int32)
    npad = E_PAD - N_EDGES
    r = jnp.arange(npad, dtype=jnp.int32)
    pad = jnp.concatenate(
        [(N_NODES + r % (N_Y - N_NODES))[None, :],
         r[None, :]], axis=0)
    ei = jnp.concatenate([ei, pad], axis=1)
    zrow = jnp.zeros((1000, D), jnp.float32)
    zdeg = jnp.zeros((1000,), jnp.float32)
    b1r = b1.reshape(1, D)
    b2r = b2.reshape(1, D)

    y1, z1, pk = _tc1(x, W1_l, W1_r, b1r, ei)
    agg1, deg = _sc_agg_deg(y1, pk, zrow, zdeg)
    deg3 = deg.reshape(NC, N_NODES, 1)
    y2, z2 = _tc2(agg1, deg3, z1, W2_l, W2_r, b2r)
    agg2 = _sc_agg(y2, pk, zrow)
    out = _tc3(agg2, deg3, z2)
    return out


# speedup vs baseline: 3.8021x; 1.0382x over previous
import functools

import jax
import jax.numpy as jnp
from jax import lax
from jax.experimental import pallas as pl
from jax.experimental.pallas import tpu as pltpu
from jax.experimental.pallas import tpu_sc as plsc

N_NODES = 10000
N_EDGES = 320000
D = 128

NC = 2
NS = 16
NW = NC * NS
E_PAD = 327680
EDGES_PER_WORKER = E_PAD // NW
CHUNK = 80
NCHUNKS = EDGES_PER_WORKER // CHUNK
PAD_CHUNK0 = (N_EDGES - (NW - 1) * EDGES_PER_WORKER) // CHUNK
N_Y = 11000

ROW_BLK = 1000
GRID = N_NODES // ROW_BLK
GRID_Y = GRID + 1

_mesh = plsc.VectorSubcoreMesh(
    core_axis_name="c", subcore_axis_name="s", num_cores=NC, num_subcores=NS
)


NB = 3


def _sc_agg_body(with_deg, *refs):
    if with_deg:
        (y_hbm, pk_hbm, zrow_hbm, zdeg_hbm,
         agg_out, deg_out,
         pk_v, sb0, sb1, sb2, db0, db1, db2, r0, r1, r2, ones_v, degstg_v,
         agg_sh, deg_sh, g0, g1, g2, s0, s1, s2) = refs
    else:
        (y_hbm, pk_hbm, zrow_hbm,
         agg_out,
         pk_v, sb0, sb1, sb2, db0, db1, db2, r0, r1, r2,
         agg_sh, g0, g1, g2, s0, s1, s2) = refs
    srcb = (sb0, sb1, sb2)
    dstb = (db0, db1, db2)
    rows = (r0, r1, r2)
    gsem = (g0, g1, g2)
    ssem = (s0, s1, s2)

    c = lax.axis_index("c")
    s = lax.axis_index("s")
    wid = c * NS + s

    @pl.when(s < 10)
    def _zero():
        pltpu.sync_copy(zrow_hbm, agg_sh.at[pl.ds(s * 1000, 1000)])
        if with_deg:
            pltpu.sync_copy(zdeg_hbm, degstg_v)
            pltpu.sync_copy(degstg_v, deg_sh.at[pl.ds(s * 1000, 1000)])

    if with_deg:
        for i in range(CHUNK // 16):
            ones_v[pl.ds(i * 16, 16)] = jnp.ones((16,), jnp.float32)

    off = pl.multiple_of(wid * EDGES_PER_WORKER, 8)
    pltpu.sync_copy(pk_hbm.at[pl.ds(off, EDGES_PER_WORKER)], pk_v)

    plsc.subcore_barrier()

    def unpack(chunk, b):
        base = chunk * CHUNK
        for k in range(CHUNK // 16):
            p = pk_v[pl.ds(base + 16 * k, 16)]
            srcb[b][pl.ds(16 * k, 16)] = p & 0xFFFF
            dstb[b][pl.ds(16 * k, 16)] = lax.shift_right_logical(p, 16)

    def fire_gather(chunk, b):
        unpack(chunk, b)
        pltpu.async_copy(y_hbm.at[srcb[b]], rows[b], gsem[b])

    def wait_gather(b):
        pltpu.make_async_copy(y_hbm.at[pl.ds(0, CHUNK)], rows[b],
                              gsem[b]).wait()

    is_last = wid == (NW - 1)

    def fire_scatter(chunk, b):
        pltpu.async_copy(rows[b], agg_sh.at[dstb[b]], ssem[b], add=True)
        if with_deg:
            @pl.when(jnp.logical_not(is_last & (chunk >= PAD_CHUNK0)))
            def _():
                pltpu.async_copy(ones_v, deg_sh.at[dstb[b]], ssem[b],
                                 add=True)

    def wait_scatter(chunk, b):
        pltpu.make_async_copy(y_hbm.at[pl.ds(0, CHUNK)], rows[b],
                              ssem[b]).wait()
        if with_deg:
            @pl.when(jnp.logical_not(is_last & (chunk >= PAD_CHUNK0)))
            def _():
                pltpu.make_async_copy(zrow_hbm.at[0, pl.ds(0, CHUNK)],
                                      ones_v, ssem[b]).wait()

    assert NCHUNKS % NB == 2
    fire_gather(0, 0)

    def body(j, carry):
        for b in range(NB):
            i = NB * j + b
            b1 = (b + 1) % NB

            @pl.when(i >= 2)
            def _w():
                wait_scatter(i - 2, b1)

            fire_gather(i + 1, b1)
            wait_gather(b)
            fire_scatter(i, b)
        return carry

    lax.fori_loop(0, NCHUNKS // NB, body, 0)
    wait_scatter(NCHUNKS - 4, 1)
    fire_gather(NCHUNKS - 1, 1)
    wait_gather(0)
    fire_scatter(NCHUNKS - 2, 0)
    wait_gather(1)
    fire_scatter(NCHUNKS - 1, 1)
    wait_scatter(NCHUNKS - 3, 2)
    wait_scatter(NCHUNKS - 2, 0)
    wait_scatter(NCHUNKS - 1, 1)

    plsc.subcore_barrier()

    @pl.when(s < 10)
    def _writeback():
        pltpu.sync_copy(agg_sh.at[pl.ds(s * 1000, 1000)],
                        agg_out.at[c, pl.ds(s * 1000, 1000)])
        if with_deg:
            off = pl.multiple_of(c * N_NODES + s * 1000, 8)
            pltpu.sync_copy(deg_sh.at[pl.ds(s * 1000, 1000)], degstg_v)
            pltpu.sync_copy(degstg_v, deg_out.at[pl.ds(off, 1000)])


_sc_agg_deg = pl.kernel(
    functools.partial(_sc_agg_body, True),
    out_type=(
        jax.ShapeDtypeStruct((NC, N_NODES, D), jnp.float32),
        jax.ShapeDtypeStruct((NC * N_NODES,), jnp.float32),
    ),
    mesh=_mesh,
    scratch_types=(
        pltpu.VMEM((EDGES_PER_WORKER,), jnp.int32),
        pltpu.VMEM((CHUNK,), jnp.int32),
        pltpu.VMEM((CHUNK,), jnp.int32),
        pltpu.VMEM((CHUNK,), jnp.int32),
        pltpu.VMEM((CHUNK,), jnp.int32),
        pltpu.VMEM((CHUNK,), jnp.int32),
        pltpu.VMEM((CHUNK,), jnp.int32),
        pltpu.VMEM((CHUNK, D), jnp.float32),
        pltpu.VMEM((CHUNK, D), jnp.float32),
        pltpu.VMEM((CHUNK, D), jnp.float32),
        pltpu.VMEM((CHUNK,), jnp.float32),
        pltpu.VMEM((1000,), jnp.float32),
        pltpu.VMEM_SHARED((N_NODES, D), jnp.float32),
        pltpu.VMEM_SHARED((N_NODES,), jnp.float32),
        pltpu.SemaphoreType.DMA,
        pltpu.SemaphoreType.DMA,
        pltpu.SemaphoreType.DMA,
        pltpu.SemaphoreType.DMA,
        pltpu.SemaphoreType.DMA,
        pltpu.SemaphoreType.DMA,
    ),
)

_sc_agg = pl.kernel(
    functools.partial(_sc_agg_body, False),
    out_type=jax.ShapeDtypeStruct((NC, N_NODES, D), jnp.float32),
    mesh=_mesh,
    scratch_types=(
        pltpu.VMEM((EDGES_PER_WORKER,), jnp.int32),
        pltpu.VMEM((CHUNK,), jnp.int32),
        pltpu.VMEM((CHUNK,), jnp.int32),
        pltpu.VMEM((CHUNK,), jnp.int32),
        pltpu.VMEM((CHUNK,), jnp.int32),
        pltpu.VMEM((CHUNK,), jnp.int32),
        pltpu.VMEM((CHUNK,), jnp.int32),
        pltpu.VMEM((CHUNK, D), jnp.float32),
        pltpu.VMEM((CHUNK, D), jnp.float32),
        pltpu.VMEM((CHUNK, D), jnp.float32),
        pltpu.VMEM_SHARED((N_NODES, D), jnp.float32),
        pltpu.SemaphoreType.DMA,
        pltpu.SemaphoreType.DMA,
        pltpu.SemaphoreType.DMA,
        pltpu.SemaphoreType.DMA,
        pltpu.SemaphoreType.DMA,
        pltpu.SemaphoreType.DMA,
    ),
)


def _tc1_body(x_ref, wl_ref, wr_ref, b_ref, ei_ref, y_ref, z_ref, pk_ref):
    pid = pl.program_id(0)

    @pl.when(pid < GRID)
    def _compute():
        xb = x_ref[...]
        y_ref[...] = jnp.dot(xb, wl_ref[...],
                             preferred_element_type=jnp.float32)
        z_ref[...] = (
            jnp.dot(xb, wr_ref[...], preferred_element_type=jnp.float32)
            + b_ref[...]
        )

    @pl.when(pid == GRID)
    def _zero_row_block():
        y_ref[...] = jnp.zeros((ROW_BLK, D), jnp.float32)

    pk_ref[...] = ei_ref[0] | (ei_ref[1] << 16)


def _tc2_body(agg_ref, deg_ref, z_ref, wl_ref, wr_ref, b_ref, y2_ref, z2_ref):
    pid = pl.program_id(0)

    @pl.when(pid < GRID)
    def _compute():
        agg = agg_ref[0] + agg_ref[1]
        dl = (deg_ref[0, pl.ds(pid, 1), :]
              + deg_ref[1, pl.ds(pid, 1), :])
        deg = jnp.maximum(jnp.transpose(dl, (1, 0)), 1.0)
        h = jnp.maximum(agg / deg + z_ref[...], 0.0)
        y2_ref[...] = jnp.dot(h, wl_ref[...],
                              preferred_element_type=jnp.float32)
        z2_ref[...] = (
            jnp.dot(h, wr_ref[...], preferred_element_type=jnp.float32)
            + b_ref[...]
        )

    @pl.when(pid == GRID)
    def _zero_row_block():
        y2_ref[...] = jnp.zeros((ROW_BLK, D), jnp.float32)


def _tc3_body(agg_ref, deg_ref, z_ref, o_ref):
    pid = pl.program_id(0)
    agg = agg_ref[0] + agg_ref[1]
    dl = deg_ref[0, pl.ds(pid, 1), :] + deg_ref[1, pl.ds(pid, 1), :]
    deg = jnp.maximum(jnp.transpose(dl, (1, 0)), 1.0)
    h = agg / deg + z_ref[...]
    m = jnp.max(h, axis=-1, keepdims=True)
    e = jnp.exp(h - m)
    lse = jnp.log(jnp.sum(e, axis=-1, keepdims=True)) + m
    o_ref[...] = h - lse


_row_spec = pl.BlockSpec((ROW_BLK, D), lambda i: (i, 0))
_row_clamp = pl.BlockSpec((ROW_BLK, D), lambda i: (jnp.minimum(i, GRID - 1), 0))
_w_spec = pl.BlockSpec((D, D), lambda i: (0, 0))
_b_spec = pl.BlockSpec((1, D), lambda i: (0, 0))
_agg_spec = pl.BlockSpec((NC, ROW_BLK, D), lambda i: (0, i, 0))
_agg_clamp = pl.BlockSpec((NC, ROW_BLK, D),
                          lambda i: (0, jnp.minimum(i, GRID - 1), 0))
_deg_spec = pl.BlockSpec((NC, GRID, ROW_BLK), lambda i: (0, 0, 0))
_deg_clamp = _deg_spec

_E_BLK = E_PAD // GRID
_tc1 = pl.pallas_call(
    _tc1_body,
    grid=(GRID_Y,),
    in_specs=[_row_clamp, _w_spec, _w_spec, _b_spec,
              pl.BlockSpec((2, _E_BLK),
                           lambda i: (0, jnp.minimum(i, GRID - 1)))],
    out_specs=[_row_spec, _row_spec,
               pl.BlockSpec((_E_BLK,),
                            lambda i: (jnp.minimum(i, GRID - 1),))],
    out_shape=[
        jax.ShapeDtypeStruct((N_Y, D), jnp.float32),
        jax.ShapeDtypeStruct((N_Y, D), jnp.float32),
        jax.ShapeDtypeStruct((E_PAD,), jnp.int32),
    ],
)

_tc2 = pl.pallas_call(
    _tc2_body,
    grid=(GRID_Y,),
    in_specs=[_agg_clamp, _deg_clamp, _row_clamp, _w_spec, _w_spec, _b_spec],
    out_specs=[_row_spec, _row_spec],
    out_shape=[
        jax.ShapeDtypeStruct((N_Y, D), jnp.float32),
        jax.ShapeDtypeStruct((N_Y, D), jnp.float32),
    ],
)

_tc3 = pl.pallas_call(
    _tc3_body,
    grid=(GRID,),
    in_specs=[_agg_spec, _deg_spec, _row_spec],
    out_specs=_row_spec,
    out_shape=jax.ShapeDtypeStruct((N_NODES, D), jnp.float32),
)


def kernel(x, edge_index, W1_l, W1_r, b1, W2_l, W2_r, b2):
    ei = edge_index.astype(jnp.int32)
    npad = E_PAD - N_EDGES
    r = jnp.arange(npad, dtype=jnp.int32)
    pad = jnp.concatenate(
        [(N_NODES + r % (N_Y - N_NODES))[None, :],
         r[None, :]], axis=0)
    ei = jnp.concatenate([ei, pad], axis=1)
    zrow = jnp.zeros((1000, D), jnp.float32)
    zdeg = jnp.zeros((1000,), jnp.float32)
    b1r = b1.reshape(1, D)
    b2r = b2.reshape(1, D)

    y1, z1, pk = _tc1(x, W1_l, W1_r, b1r, ei)
    agg1, deg = _sc_agg_deg(y1, pk, zrow, zdeg)
    deg3 = deg.reshape(NC, GRID, ROW_BLK)
    y2, z2 = _tc2(agg1, deg3, z1, W2_l, W2_r, b2r)
    agg2 = _sc_agg(y2, pk, zrow)
    out = _tc3(agg2, deg3, z2)
    return out


# speedup vs baseline: 3.8689x; 1.0176x over previous
import functools

import jax
import jax.numpy as jnp
from jax import lax
from jax.experimental import pallas as pl
from jax.experimental.pallas import tpu as pltpu
from jax.experimental.pallas import tpu_sc as plsc

N_NODES = 10000
N_EDGES = 320000
D = 128

NC = 2
NS = 16
NW = NC * NS
E_PAD = 327680
EDGES_PER_WORKER = E_PAD // NW
CHUNK = 80
NCHUNKS = EDGES_PER_WORKER // CHUNK
PAD_CHUNK0 = (N_EDGES - (NW - 1) * EDGES_PER_WORKER) // CHUNK
N_Y = 11000

ROW_BLK = 1000
GRID = N_NODES // ROW_BLK
GRID_Y = GRID + 1

_mesh = plsc.VectorSubcoreMesh(
    core_axis_name="c", subcore_axis_name="s", num_cores=NC, num_subcores=NS
)


NB = 3


def _sc_agg_body(with_deg, *refs):
    if with_deg:
        (y_hbm, pk_hbm, zrow_hbm, zdeg_hbm,
         agg_out, deg_out,
         pk_v, sb0, sb1, sb2, db0, db1, db2, r0, r1, r2, ones_v, degstg_v,
         agg_sh, deg_sh, g0, g1, g2, s0, s1, s2) = refs
    else:
        (y_hbm, pk_hbm, zrow_hbm,
         agg_out,
         pk_v, sb0, sb1, sb2, db0, db1, db2, r0, r1, r2,
         agg_sh, g0, g1, g2, s0, s1, s2) = refs
    srcb = (sb0, sb1, sb2)
    dstb = (db0, db1, db2)
    rows = (r0, r1, r2)
    gsem = (g0, g1, g2)
    ssem = (s0, s1, s2)

    c = lax.axis_index("c")
    s = lax.axis_index("s")
    wid = c * NS + s

    @pl.when(s < 10)
    def _zero():
        pltpu.sync_copy(zrow_hbm, agg_sh.at[pl.ds(s * 1000, 1000)])
        if with_deg:
            pltpu.sync_copy(zdeg_hbm, degstg_v)
            pltpu.sync_copy(degstg_v, deg_sh.at[pl.ds(s * 1000, 1000)])

    if with_deg:
        for i in range(CHUNK // 16):
            ones_v[pl.ds(i * 16, 16)] = jnp.ones((16,), jnp.float32)

    off = pl.multiple_of(wid * EDGES_PER_WORKER, 8)
    pltpu.sync_copy(pk_hbm.at[pl.ds(off, EDGES_PER_WORKER)], pk_v)

    plsc.subcore_barrier()

    def unpack(chunk, b):
        base = chunk * CHUNK
        for k in range(CHUNK // 16):
            p = pk_v[pl.ds(base + 16 * k, 16)]
            srcb[b][pl.ds(16 * k, 16)] = p & 0xFFFF
            dstb[b][pl.ds(16 * k, 16)] = lax.shift_right_logical(p, 16)

    def fire_gather(chunk, b):
        unpack(chunk, b)
        pltpu.async_copy(y_hbm.at[srcb[b]], rows[b], gsem[b])

    def wait_gather(b):
        pltpu.make_async_copy(y_hbm.at[pl.ds(0, CHUNK)], rows[b],
                              gsem[b]).wait()

    is_last = wid == (NW - 1)

    def fire_scatter(chunk, b):
        pltpu.async_copy(rows[b], agg_sh.at[dstb[b]], ssem[b], add=True)
        if with_deg:
            @pl.when(jnp.logical_not(is_last & (chunk >= PAD_CHUNK0)))
            def _():
                pltpu.async_copy(ones_v, deg_sh.at[dstb[b]], ssem[b],
                                 add=True)

    def wait_scatter(chunk, b):
        pltpu.make_async_copy(y_hbm.at[pl.ds(0, CHUNK)], rows[b],
                              ssem[b]).wait()
        if with_deg:
            @pl.when(jnp.logical_not(is_last & (chunk >= PAD_CHUNK0)))
            def _():
                pltpu.make_async_copy(zrow_hbm.at[0, pl.ds(0, CHUNK)],
                                      ones_v, ssem[b]).wait()

    assert NCHUNKS % NB == 2
    fire_gather(0, 0)

    def body(j, carry):
        for b in range(NB):
            i = NB * j + b
            b1 = (b + 1) % NB

            @pl.when(i >= 2)
            def _w():
                wait_scatter(i - 2, b1)

            fire_gather(i + 1, b1)
            wait_gather(b)
            fire_scatter(i, b)
        return carry

    lax.fori_loop(0, NCHUNKS // NB, body, 0)
    wait_scatter(NCHUNKS - 4, 1)
    fire_gather(NCHUNKS - 1, 1)
    wait_gather(0)
    fire_scatter(NCHUNKS - 2, 0)
    wait_gather(1)
    fire_scatter(NCHUNKS - 1, 1)
    wait_scatter(NCHUNKS - 3, 2)
    wait_scatter(NCHUNKS - 2, 0)
    wait_scatter(NCHUNKS - 1, 1)

    plsc.subcore_barrier()

    @pl.when(s < 10)
    def _writeback():
        pltpu.sync_copy(agg_sh.at[pl.ds(s * 1000, 1000)],
                        agg_out.at[c, pl.ds(s * 1000, 1000)])
        if with_deg:
            off = pl.multiple_of(c * N_NODES + s * 1000, 8)
            pltpu.sync_copy(deg_sh.at[pl.ds(s * 1000, 1000)], degstg_v)
            pltpu.sync_copy(degstg_v, deg_out.at[pl.ds(off, 1000)])


_sc_agg_deg = pl.kernel(
    functools.partial(_sc_agg_body, True),
    out_type=(
        jax.ShapeDtypeStruct((NC, N_NODES, D), jnp.float32),
        jax.ShapeDtypeStruct((NC * N_NODES,), jnp.float32),
    ),
    mesh=_mesh,
    scratch_types=(
        pltpu.VMEM((EDGES_PER_WORKER,), jnp.int32),
        pltpu.VMEM((CHUNK,), jnp.int32),
        pltpu.VMEM((CHUNK,), jnp.int32),
        pltpu.VMEM((CHUNK,), jnp.int32),
        pltpu.VMEM((CHUNK,), jnp.int32),
        pltpu.VMEM((CHUNK,), jnp.int32),
        pltpu.VMEM((CHUNK,), jnp.int32),
        pltpu.VMEM((CHUNK, D), jnp.float32),
        pltpu.VMEM((CHUNK, D), jnp.float32),
        pltpu.VMEM((CHUNK, D), jnp.float32),
        pltpu.VMEM((CHUNK,), jnp.float32),
        pltpu.VMEM((1000,), jnp.float32),
        pltpu.VMEM_SHARED((N_NODES, D), jnp.float32),
        pltpu.VMEM_SHARED((N_NODES,), jnp.float32),
        pltpu.SemaphoreType.DMA,
        pltpu.SemaphoreType.DMA,
        pltpu.SemaphoreType.DMA,
        pltpu.SemaphoreType.DMA,
        pltpu.SemaphoreType.DMA,
        pltpu.SemaphoreType.DMA,
    ),
)

_sc_agg = pl.kernel(
    functools.partial(_sc_agg_body, False),
    out_type=jax.ShapeDtypeStruct((NC, N_NODES, D), jnp.float32),
    mesh=_mesh,
    scratch_types=(
        pltpu.VMEM((EDGES_PER_WORKER,), jnp.int32),
        pltpu.VMEM((CHUNK,), jnp.int32),
        pltpu.VMEM((CHUNK,), jnp.int32),
        pltpu.VMEM((CHUNK,), jnp.int32),
        pltpu.VMEM((CHUNK,), jnp.int32),
        pltpu.VMEM((CHUNK,), jnp.int32),
        pltpu.VMEM((CHUNK,), jnp.int32),
        pltpu.VMEM((CHUNK, D), jnp.float32),
        pltpu.VMEM((CHUNK, D), jnp.float32),
        pltpu.VMEM((CHUNK, D), jnp.float32),
        pltpu.VMEM_SHARED((N_NODES, D), jnp.float32),
        pltpu.SemaphoreType.DMA,
        pltpu.SemaphoreType.DMA,
        pltpu.SemaphoreType.DMA,
        pltpu.SemaphoreType.DMA,
        pltpu.SemaphoreType.DMA,
        pltpu.SemaphoreType.DMA,
    ),
)


def _tc1_body(x_ref, wl_ref, wr_ref, b_ref, ei_ref, y_ref, z_ref, pk_ref):
    pid = pl.program_id(0)

    @pl.when(pid < GRID)
    def _compute():
        xb = x_ref[...]
        y_ref[...] = jnp.dot(xb, wl_ref[...],
                             preferred_element_type=jnp.float32)
        z_ref[...] = (
            jnp.dot(xb, wr_ref[...], preferred_element_type=jnp.float32)
            + b_ref[...]
        )

    @pl.when(pid == GRID)
    def _zero_row_block():
        y_ref[...] = jnp.zeros((ROW_BLK, D), jnp.float32)

    gi0 = jnp.minimum(pid, GRID - 1) * _E_BLK
    pos = gi0 + lax.broadcasted_iota(jnp.int32, (1, _E_BLK), 1)
    sv = ei_ref[0:1, :]
    dv = ei_ref[1:2, :]
    r = jnp.maximum(pos - N_EDGES, 0)
    pad_pk = (N_NODES + r % (N_Y - N_NODES)) | (r << 16)
    pkv = jnp.where(pos < N_EDGES, sv | (dv << 16), pad_pk)
    pk_ref[...] = jnp.reshape(pkv, (_E_BLK,))


def _tc2_body(agg_ref, deg_ref, z_ref, wl_ref, wr_ref, b_ref, y2_ref, z2_ref):
    pid = pl.program_id(0)

    @pl.when(pid < GRID)
    def _compute():
        agg = agg_ref[0] + agg_ref[1]
        dl = (deg_ref[0, pl.ds(pid, 1), :]
              + deg_ref[1, pl.ds(pid, 1), :])
        deg = jnp.maximum(jnp.transpose(dl, (1, 0)), 1.0)
        h = jnp.maximum(agg / deg + z_ref[...], 0.0)
        y2_ref[...] = jnp.dot(h, wl_ref[...],
                              preferred_element_type=jnp.float32)
        z2_ref[...] = (
            jnp.dot(h, wr_ref[...], preferred_element_type=jnp.float32)
            + b_ref[...]
        )

    @pl.when(pid == GRID)
    def _zero_row_block():
        y2_ref[...] = jnp.zeros((ROW_BLK, D), jnp.float32)


def _tc3_body(agg_ref, deg_ref, z_ref, o_ref):
    pid = pl.program_id(0)
    agg = agg_ref[0] + agg_ref[1]
    dl = deg_ref[0, pl.ds(pid, 1), :] + deg_ref[1, pl.ds(pid, 1), :]
    deg = jnp.maximum(jnp.transpose(dl, (1, 0)), 1.0)
    h = agg / deg + z_ref[...]
    m = jnp.max(h, axis=-1, keepdims=True)
    e = jnp.exp(h - m)
    lse = jnp.log(jnp.sum(e, axis=-1, keepdims=True)) + m
    o_ref[...] = h - lse


_row_spec = pl.BlockSpec((ROW_BLK, D), lambda i: (i, 0))
_row_clamp = pl.BlockSpec((ROW_BLK, D), lambda i: (jnp.minimum(i, GRID - 1), 0))
_w_spec = pl.BlockSpec((D, D), lambda i: (0, 0))
_b_spec = pl.BlockSpec((1, D), lambda i: (0, 0))
_agg_spec = pl.BlockSpec((NC, ROW_BLK, D), lambda i: (0, i, 0))
_agg_clamp = pl.BlockSpec((NC, ROW_BLK, D),
                          lambda i: (0, jnp.minimum(i, GRID - 1), 0))
_deg_spec = pl.BlockSpec((NC, GRID, ROW_BLK), lambda i: (0, 0, 0))
_deg_clamp = _deg_spec

_E_BLK = E_PAD // GRID
_tc1 = pl.pallas_call(
    _tc1_body,
    grid=(GRID_Y,),
    in_specs=[_row_clamp, _w_spec, _w_spec, _b_spec,
              pl.BlockSpec((2, _E_BLK),
                           lambda i: (0, jnp.minimum(i, GRID - 1)))],
    out_specs=[_row_spec, _row_spec,
               pl.BlockSpec((_E_BLK,),
                            lambda i: (jnp.minimum(i, GRID - 1),))],
    out_shape=[
        jax.ShapeDtypeStruct((N_Y, D), jnp.float32),
        jax.ShapeDtypeStruct((N_Y, D), jnp.float32),
        jax.ShapeDtypeStruct((E_PAD,), jnp.int32),
    ],
)

_tc2 = pl.pallas_call(
    _tc2_body,
    grid=(GRID_Y,),
    in_specs=[_agg_clamp, _deg_clamp, _row_clamp, _w_spec, _w_spec, _b_spec],
    out_specs=[_row_spec, _row_spec],
    out_shape=[
        jax.ShapeDtypeStruct((N_Y, D), jnp.float32),
        jax.ShapeDtypeStruct((N_Y, D), jnp.float32),
    ],
)

_tc3 = pl.pallas_call(
    _tc3_body,
    grid=(GRID,),
    in_specs=[_agg_spec, _deg_spec, _row_spec],
    out_specs=_row_spec,
    out_shape=jax.ShapeDtypeStruct((N_NODES, D), jnp.float32),
)


def kernel(x, edge_index, W1_l, W1_r, b1, W2_l, W2_r, b2):
    ei = edge_index.astype(jnp.int32)
    zrow = jnp.zeros((1000, D), jnp.float32)
    zdeg = jnp.zeros((1000,), jnp.float32)
    b1r = b1.reshape(1, D)
    b2r = b2.reshape(1, D)

    y1, z1, pk = _tc1(x, W1_l, W1_r, b1r, ei)
    agg1, deg = _sc_agg_deg(y1, pk, zrow, zdeg)
    deg3 = deg.reshape(NC, GRID, ROW_BLK)
    y2, z2 = _tc2(agg1, deg3, z1, W2_l, W2_r, b2r)
    agg2 = _sc_agg(y2, pk, zrow)
    out = _tc3(agg2, deg3, z2)
    return out
